# baseline (speedup 1.0000x reference)
import jax
import jax.numpy as jnp
from jax import lax
from jax.experimental import pallas as pl
from jax.experimental.pallas import tpu as pltpu
from jax.experimental.pallas import tpu_sc as plsc

NN = 10000
NE = 320000
F = 128
FH = 64
WIDA = 72
NC = 2
NS = 16
EPW = 20480
NEP = NS * EPW
K = 64
NB = EPW // K
GBUF = 4
CH = 8
NCH = NB // CH
ROWS = 10048
ZNS = 8
ZSTRIPE = ROWS // ZNS


def _splat(x):
    return lax.broadcast(x, (16,))


def _tc_pack(e_ref, src_ref, dst_ref):
    for r in range(NS):
        lo = r * EPW
        n = min(NE - lo, EPW)
        src_ref[pl.ds(r, 1), pl.ds(0, n)] = e_ref[pl.ds(0, 1), pl.ds(lo, n)]
        dst_ref[pl.ds(r, 1), pl.ds(0, n)] = e_ref[pl.ds(1, 1), pl.ds(lo, n)]
        if n < EPW:
            src_ref[pl.ds(r, 1), pl.ds(n, EPW - n)] = jnp.zeros(
                (1, EPW - n), jnp.int32)
            dst_ref[pl.ds(r, 1), pl.ds(n, EPW - n)] = jnp.full(
                (1, EPW - n), NN, jnp.int32)


def _tc_pre(h_ref, w1t_ref, b1_ref, w2t_ref, b2_ref, u_ref, p_ref):
    hb = h_ref[...]
    u = jax.lax.dot_general(hb, w1t_ref[...], (((1,), (0,)), ((), ())),
                            precision=lax.Precision.HIGHEST,
                            preferred_element_type=jnp.float32)
    u_ref[...] = u + b1_ref[...]
    p = jax.lax.dot_general(hb, w2t_ref[...], (((1,), (0,)), ((), ())),
                            precision=lax.Precision.HIGHEST,
                            preferred_element_type=jnp.float32)
    p_ref[...] = p + b2_ref[...]


def _tc_post(a_ref, p_ref, w2bt_ref, o_ref):
    cnt = jnp.maximum(a_ref[0, :, FH:FH + 1], 1.0)
    h_n = jnp.concatenate([a_ref[0, :, :FH], a_ref[1, :, :FH]], axis=1) / cnt
    acc = jax.lax.dot_general(h_n, w2bt_ref[...], (((1,), (0,)), ((), ())),
                              precision=lax.Precision.HIGHEST,
                              preferred_element_type=jnp.float32)
    o_ref[...] = jnp.maximum(p_ref[...] + acc, 0.0)


def _sc_edge(u_hbm, src_hbm, dst_hbm, w_hbm, w1w_hbm, acc_hbm,
             sbuf, dbuf, wbuf, w1wv, gbuf, mbuf, acc_s, u_s,
             csem, gsem, ssem):
    cid = lax.axis_index("c")
    sid = lax.axis_index("s")

    pltpu.sync_copy(w1w_hbm.at[cid], w1wv)

    @pl.when(sid < 5)
    def _stage_u():
        ub = pl.multiple_of(sid * 2000, 8)
        cb = pl.multiple_of(cid * FH, 8)
        pltpu.sync_copy(u_hbm.at[pl.ds(ub, 2000), pl.ds(cb, FH)],
                        u_s.at[pl.ds(ub, 2000)])

    zv = jnp.zeros((16,), jnp.float32)

    @pl.loop(0, K)
    def _zrow(r):
        for j in range(WIDA // 16):
            mbuf[0, r, pl.ds(16 * j, 16)] = zv
        mbuf[0, r, pl.ds(WIDA - 16, 16)] = zv

    @pl.when(sid < ZNS)
    def _zero_acc():
        base = pl.multiple_of(sid * ZSTRIPE, 8)
        for i in range(ZSTRIPE // K):
            pltpu.sync_copy(mbuf.at[0], acc_s.at[pl.ds(base + i * K, K)])
        rem = ZSTRIPE % K
        if rem:
            pltpu.sync_copy(mbuf.at[0, pl.ds(0, rem)],
                            acc_s.at[pl.ds(base + (ZSTRIPE // K) * K, rem)])

    lane = lax.broadcasted_iota(jnp.int32, (16,), 0)
    cvec = jnp.where(lane == FH - (WIDA - 16), 1.0, 0.0)

    @pl.loop(0, K)
    def _crow(r):
        for sl in range(GBUF):
            mbuf[sl, r, pl.ds(WIDA - 16, 16)] = cvec

    plsc.subcore_barrier()

    cs = [[w1wv[ci, pl.ds(16 * j, 16)] for j in range(FH // 16)]
          for ci in range(3)]

    def cdesc(c, sl):
        return (pltpu.make_async_copy(src_hbm.at[sid, c], sbuf.at[sl],
                                      csem.at[sl]),
                pltpu.make_async_copy(dst_hbm.at[sid, c], dbuf.at[sl],
                                      csem.at[sl]),
                pltpu.make_async_copy(w_hbm.at[sid, c], wbuf.at[sl],
                                      csem.at[sl]))

    def gdesc(hh, j, sg):
        return pltpu.make_async_copy(
            u_s.at[sbuf.at[hh, j]], gbuf.at[sg], gsem.at[sg])

    def sdesc(hh, j, sm):
        return pltpu.make_async_copy(
            mbuf.at[sm], acc_s.at[dbuf.at[hh, j]], ssem.at[sm])

    for d in cdesc(0, 0):
        d.start()
    for d in cdesc(0, 0):
        d.wait()
    gdesc(0, 0, 0).start()
    gdesc(0, 1, 1).start()

    @pl.loop(0, NB, step=2 * CH)
    def _outer(b0):
        c0 = b0 // CH
        for kk in range(2 * CH):
            b = b0 + kk
            h = kk // CH
            j = kk % CH
            sm = kk % GBUF

            hb4 = ((kk - 4) % (2 * CH)) // CH
            jb4 = (kk - 4) % CH
            if kk >= 4:
                sdesc(hb4, jb4, sm).wait()
            else:
                @pl.when(b >= 4)
                def _():
                    sdesc(hb4, jb4, sm).wait()

            if kk == 4:
                for d in cdesc(c0 + 1, 1):
                    d.start()
            if kk == CH + 4:
                @pl.when(b0 + 2 * CH < NB)
                def _():
                    for d in cdesc(c0 + 2, 0):
                        d.start()

            j2 = (kk + 2) % CH
            h2 = ((kk + 2) % (2 * CH)) // CH
            sg2 = (kk + 2) % GBUF
            if kk == CH - 2:
                for d in cdesc(c0 + 1, 1):
                    d.wait()
                gdesc(h2, j2, sg2).start()
            elif kk == 2 * CH - 2:
                @pl.when(b + 2 < NB)
                def _():
                    for d in cdesc(c0 + 2, 0):
                        d.wait()
                    gdesc(h2, j2, sg2).start()
            elif kk == 2 * CH - 1:
                @pl.when(b + 2 < NB)
                def _():
                    gdesc(h2, j2, sg2).start()
            else:
                gdesc(h2, j2, sg2).start()

            gdesc(h, j, sm).wait()

            @plsc.parallel_loop(0, K, unroll=2)
            def _edge(e):
                h_v = _splat(h)
                j_v = _splat(j)
                e_v = _splat(e)
                ws = [plsc.load_gather(wbuf, [h_v, j_v, e_v, _splat(ci)])
                      for ci in range(3)]
                for g in range(FH // 16):
                    t = gbuf[sm, e, pl.ds(16 * g, 16)]
                    t = t + ws[0] * cs[0][g] + ws[1] * cs[1][g] \
                        + ws[2] * cs[2][g]
                    t = jnp.maximum(t, t * 0.01)
                    mbuf[sm, e, pl.ds(16 * g, 16)] = t

            sdesc(h, j, sm).start(add=True)

    for kk in range(2 * CH - 4, 2 * CH):
        sdesc(kk // CH, kk % CH, kk % GBUF).wait()
    plsc.subcore_barrier()

    @pl.when(sid < ZNS)
    def _writeout():
        base = pl.multiple_of(sid * ZSTRIPE, 8)
        pltpu.sync_copy(acc_s.at[pl.ds(base, ZSTRIPE)],
                        acc_hbm.at[cid, pl.ds(base, ZSTRIPE)])


@jax.jit
def kernel(h, edge_index, w, W1, b1, W2, b2):
    wp = jnp.concatenate([w.astype(jnp.float32),
                          jnp.zeros((NEP - NE, 3), jnp.float32)])
    src_p, dst_p = pl.pallas_call(
        _tc_pack,
        in_specs=[pl.BlockSpec(memory_space=pltpu.VMEM)],
        out_specs=[pl.BlockSpec(memory_space=pltpu.VMEM),
                   pl.BlockSpec(memory_space=pltpu.VMEM)],
        out_shape=[jax.ShapeDtypeStruct((NS, EPW), jnp.int32),
                   jax.ShapeDtypeStruct((NS, EPW), jnp.int32)],
    )(edge_index.astype(jnp.int32))
    w1w = W1[:, F:].T.astype(jnp.float32)
    w1w_halves = jnp.stack([w1w[:, :FH], w1w[:, FH:]])

    blk = 1000
    grid = NN // blk
    pblk = 2000
    u_pad, p = pl.pallas_call(
        _tc_pre,
        grid=(NN // pblk,),
        in_specs=[
            pl.BlockSpec((pblk, F), lambda i: (i, 0)),
            pl.BlockSpec((F, F), lambda i: (0, 0)),
            pl.BlockSpec((1, F), lambda i: (0, 0)),
            pl.BlockSpec((F, F), lambda i: (0, 0)),
            pl.BlockSpec((1, F), lambda i: (0, 0)),
        ],
        out_specs=[
            pl.BlockSpec((pblk, F), lambda i: (i, 0)),
            pl.BlockSpec((pblk, F), lambda i: (i, 0)),
        ],
        out_shape=[
            jax.ShapeDtypeStruct((NN, F), jnp.float32),
            jax.ShapeDtypeStruct((NN, F), jnp.float32),
        ],
    )(h, W1[:, :F].T, b1.reshape(1, F), W2[:, :F].T, b2.reshape(1, F))

    mesh = plsc.VectorSubcoreMesh(core_axis_name="c", subcore_axis_name="s")
    acc = pl.kernel(
        _sc_edge,
        out_type=jax.ShapeDtypeStruct((NC, ROWS, WIDA), jnp.float32),
        mesh=mesh,
        compiler_params=pltpu.CompilerParams(use_tc_tiling_on_sc=False,
                                             needs_layout_passes=False),
        scratch_types=[
            pltpu.VMEM((2, CH, K), jnp.int32),
            pltpu.VMEM((2, CH, K), jnp.int32),
            pltpu.VMEM((2, CH, K, 3), jnp.float32),
            pltpu.VMEM((3, FH), jnp.float32),
            pltpu.VMEM((GBUF, K, FH), jnp.float32),
            pltpu.VMEM((GBUF, K, WIDA), jnp.float32),
            pltpu.VMEM_SHARED((ROWS, WIDA), jnp.float32),
            pltpu.VMEM_SHARED((NN, FH), jnp.float32),
            pltpu.SemaphoreType.DMA((2,)),
            pltpu.SemaphoreType.DMA((GBUF,)),
            pltpu.SemaphoreType.DMA((GBUF,)),
        ],
    )(u_pad, src_p.reshape(NS, NCH, CH, K), dst_p.reshape(NS, NCH, CH, K),
      wp.reshape(NS, NCH, CH, K, 3), w1w_halves)

    out = pl.pallas_call(
        _tc_post,
        grid=(grid,),
        in_specs=[
            pl.BlockSpec((NC, blk, WIDA), lambda i: (0, i, 0)),
            pl.BlockSpec((blk, F), lambda i: (i, 0)),
            pl.BlockSpec((F, F), lambda i: (0, 0)),
        ],
        out_specs=pl.BlockSpec((blk, F), lambda i: (i, 0)),
        out_shape=jax.ShapeDtypeStruct((NN, F), jnp.float32),
    )(acc, p, W2[:, F:].T)
    return out
